# Initial kernel scaffold; baseline (speedup 1.0000x reference)
import jax
import jax.numpy as jnp
from jax.experimental import pallas as pl


def kernel(input_ids, embedding_table, fc_weight, fc_bias):
    raise NotImplementedError("write your pallas kernel here")


# speedup vs baseline: 19.4814x; 19.4814x over previous
import functools

import jax
import jax.numpy as jnp
from jax import lax
from jax.experimental import pallas as pl
from jax.experimental.pallas import tpu as pltpu
from jax.experimental.pallas import tpu_sc as plsc

_VOCAB = 100000
_HIDDEN = 128
_BATCH = 4096
_SEQ = 200
_PAD = 16
_NW = 32
_BPW = _BATCH // _NW
_ROW_BLK = 2000


def _proj_body(table_ref, w_ref, b_ref, out_ref):
    out_ref[...] = (
        jnp.dot(table_ref[...], w_ref[...], preferred_element_type=jnp.float32)
        + b_ref[...]
    )


def _project_table(table, w_pad, b_pad):
    grid = _VOCAB // _ROW_BLK
    return pl.pallas_call(
        _proj_body,
        grid=(grid,),
        in_specs=[
            pl.BlockSpec((_ROW_BLK, _HIDDEN), lambda i: (i, 0)),
            pl.BlockSpec((_HIDDEN, _PAD), lambda i: (0, 0)),
            pl.BlockSpec((1, _PAD), lambda i: (0, 0)),
        ],
        out_specs=pl.BlockSpec((_ROW_BLK, _PAD), lambda i: (i, 0)),
        out_shape=jax.ShapeDtypeStruct((_VOCAB, _PAD), jnp.float32),
    )(table, w_pad, b_pad)


@functools.partial(
    pl.kernel,
    out_type=jax.ShapeDtypeStruct((_BATCH, _PAD), jnp.float32),
    mesh=plsc.VectorSubcoreMesh(core_axis_name="c", subcore_axis_name="s"),
    scratch_types=[
        pltpu.VMEM((_SEQ, _BPW), jnp.int32),
        pltpu.VMEM((_BPW, _PAD), jnp.float32),
        pltpu.SemaphoreType.DMA,
        pltpu.SemaphoreType.DMA,
    ],
    compiler_params=pltpu.CompilerParams(use_tc_tiling_on_sc=False),
)
def _sc_pool(ids_hbm, tp_hbm, out_hbm, idx_v, acc_v, sem0, sem):
    wid = lax.axis_index("s") * 2 + lax.axis_index("c")
    base = wid * _BPW

    pltpu.sync_copy(ids_hbm.at[wid], idx_v)

    pltpu.async_copy(tp_hbm.at[idx_v.at[0]], acc_v, sem0).wait()

    @pl.loop(1, _SEQ)
    def _issue(l):
        pltpu.async_copy(tp_hbm.at[idx_v.at[l]], acc_v, sem, add=True)

    @pl.loop(1, _SEQ)
    def _drain(l):
        pltpu.make_async_copy(tp_hbm.at[idx_v.at[0]], acc_v, sem).wait()

    pltpu.sync_copy(acc_v, out_hbm.at[pl.ds(base, _BPW)])


def kernel(input_ids, embedding_table, fc_weight, fc_bias):
    ids = input_ids.astype(jnp.int32)
    ids_t = ids.reshape(_NW, _BPW, _SEQ).transpose(0, 2, 1)

    w_pad = jnp.zeros((_HIDDEN, _PAD), jnp.float32)
    w_pad = w_pad.at[:, : fc_weight.shape[0]].set(fc_weight.T / _SEQ)
    b_pad = jnp.zeros((1, _PAD), jnp.float32)
    b_pad = b_pad.at[0, : fc_bias.shape[0]].set(fc_bias / _SEQ)

    tp = _project_table(embedding_table, w_pad, b_pad)
    out16 = _sc_pool(ids_t, tp)
    return out16[:, : fc_weight.shape[0]]


# speedup vs baseline: 20.4639x; 1.0504x over previous
import functools

import jax
import jax.numpy as jnp
from jax import lax
from jax.experimental import pallas as pl
from jax.experimental.pallas import tpu as pltpu
from jax.experimental.pallas import tpu_sc as plsc

_VOCAB = 100000
_HIDDEN = 128
_BATCH = 4096
_SEQ = 200
_PAD = 16
_NW = 32
_BPW = _BATCH // _NW
_NBUF = 16
_ROW_BLK = 4000


def _proj_body(table_ref, w_ref, b_ref, out_ref):
    out_ref[...] = (
        jnp.dot(table_ref[...], w_ref[...], preferred_element_type=jnp.float32)
        + b_ref[...]
    )


def _project_table(table, w_pad, b_pad):
    grid = _VOCAB // _ROW_BLK
    return pl.pallas_call(
        _proj_body,
        grid=(grid,),
        in_specs=[
            pl.BlockSpec((_ROW_BLK, _HIDDEN), lambda i: (i, 0)),
            pl.BlockSpec((_HIDDEN, _PAD), lambda i: (0, 0)),
            pl.BlockSpec((1, _PAD), lambda i: (0, 0)),
        ],
        out_specs=pl.BlockSpec((_ROW_BLK, _PAD), lambda i: (i, 0)),
        out_shape=jax.ShapeDtypeStruct((_VOCAB, _PAD), jnp.float32),
    )(table, w_pad, b_pad)


@functools.partial(
    pl.kernel,
    out_type=jax.ShapeDtypeStruct((_BATCH, _PAD), jnp.float32),
    mesh=plsc.VectorSubcoreMesh(core_axis_name="c", subcore_axis_name="s"),
    scratch_types=[
        pltpu.VMEM((_BPW, _SEQ), jnp.int32),
        pltpu.VMEM((_NBUF, _BPW), jnp.int32),
        pltpu.VMEM((_BPW, _PAD), jnp.float32),
        pltpu.SemaphoreType.DMA,
        pltpu.SemaphoreType.DMA,
    ],
    compiler_params=pltpu.CompilerParams(
        use_tc_tiling_on_sc=False, needs_layout_passes=False
    ),
)
def _sc_pool(ids_hbm, tp_hbm, out_hbm, ids_v, ring_v, acc_v, sem0, sem):
    wid = lax.axis_index("s") * 2 + lax.axis_index("c")
    base = wid * _BPW

    pltpu.sync_copy(ids_hbm.at[pl.ds(base, _BPW)], ids_v)

    lane = lax.iota(jnp.int32, 16)

    def build_col(l, slot):
        cols = jnp.full((16,), 0, jnp.int32) + l
        dst = ring_v.at[slot]
        for g in range(_BPW // 16):
            rows = g * 16 + lane
            dst[pl.ds(g * 16, 16)] = plsc.load_gather(ids_v, [rows, cols])

    build_col(0, 0)
    pltpu.async_copy(tp_hbm.at[ring_v.at[0]], acc_v, sem0).wait()

    @pl.loop(1, _SEQ)
    def _issue(l):
        @pl.when(l > _NBUF)
        def _():
            pltpu.make_async_copy(tp_hbm.at[ring_v.at[0]], acc_v, sem).wait()

        slot = lax.rem(l, _NBUF)
        build_col(l, slot)
        pltpu.async_copy(tp_hbm.at[ring_v.at[slot]], acc_v, sem, add=True)

    @pl.loop(0, _NBUF)
    def _drain(_):
        pltpu.make_async_copy(tp_hbm.at[ring_v.at[0]], acc_v, sem).wait()

    pltpu.sync_copy(acc_v, out_hbm.at[pl.ds(base, _BPW)])


def kernel(input_ids, embedding_table, fc_weight, fc_bias):
    ids = input_ids.astype(jnp.int32)

    w_pad = jnp.zeros((_HIDDEN, _PAD), jnp.float32)
    w_pad = w_pad.at[:, : fc_weight.shape[0]].set(fc_weight.T / _SEQ)
    b_pad = jnp.zeros((1, _PAD), jnp.float32)
    b_pad = b_pad.at[0, : fc_bias.shape[0]].set(fc_bias / _SEQ)

    tp = _project_table(embedding_table, w_pad, b_pad)
    out16 = _sc_pool(ids, tp)
    return out16[:, : fc_weight.shape[0]]
